# baseline (speedup 1.0000x reference)
import functools
import math

import jax
import jax.numpy as jnp
from jax import lax
from jax.experimental import pallas as pl
from jax.experimental.pallas import tpu as pltpu
from jax.experimental.pallas import tpu_sc as plsc


def _sc_kernel(B, F, h, w):
    hw = h * w
    C = 2 * F
    NC, NS, L = 2, 16, 16
    n_cg = 16
    n_bg = 2
    c_per = C // n_cg
    b_per = B // n_bg
    assert c_per * n_cg == C and b_per * n_bg == B
    assert w % L == 0 and h % L == 0
    tab_stage = c_per * w
    n_x_chunks = w // L
    mesh = plsc.VectorSubcoreMesh(core_axis_name="c", subcore_axis_name="s")

    @functools.partial(
        pl.kernel,
        mesh=mesh,
        out_type=jax.ShapeDtypeStruct((B, C, h, w), jnp.float32),
        scratch_types=[
            pltpu.VMEM((tab_stage,), jnp.float32),
            pltpu.VMEM((c_per, h, w), jnp.float32),
            pltpu.SemaphoreType.DMA,
        ],
    )
    def k(rowT_hbm, colT_hbm, out_hbm, tab_v, buf_v, sem):
        core = lax.axis_index("c")
        sub = lax.axis_index("s")
        cg_local = sub // n_bg
        bgroup = sub % n_bg

        t_lo = cg_local * tab_stage

        @pl.when(core == 0)
        def _():
            pltpu.sync_copy(colT_hbm.at[pl.ds(t_lo, tab_stage)],
                            tab_v.at[pl.ds(0, tab_stage)])

        @pl.when(core == 1)
        def _():
            pltpu.sync_copy(rowT_hbm.at[pl.ds(t_lo, tab_stage)],
                            tab_v.at[pl.ds(0, tab_stage)])

        @pl.when(core == 0)
        def _():
            def jloop(j, _):
                chunks = [tab_v[pl.ds(j * w + q * L, L)] for q in range(n_x_chunks)]
                for y in range(h):
                    for q in range(n_x_chunks):
                        buf_v[j, y, pl.ds(q * L, L)] = chunks[q]
                return 0

            lax.fori_loop(0, c_per, jloop, 0)

        @pl.when(core == 1)
        def _():
            def jloop(j, _):
                yvecs = [tab_v[pl.ds(j * h + q * L, L)] for q in range(h // L)]
                for y in range(h):
                    val = jnp.full((L,), yvecs[y // L][y % L], jnp.float32)
                    for q in range(n_x_chunks):
                        buf_v[j, y, pl.ds(q * L, L)] = val
                return 0

            lax.fori_loop(0, c_per, jloop, 0)

        ch_lo = core * F + cg_local * c_per
        b_lo = bgroup * b_per
        pltpu.make_async_copy(
            buf_v, out_hbm.at[b_lo, pl.ds(ch_lo, c_per)], sem
        ).start()
        pltpu.make_async_copy(
            buf_v, out_hbm.at[b_lo, pl.ds(ch_lo, c_per)], sem
        ).wait()
        src = out_hbm.at[b_lo, pl.ds(ch_lo, c_per)]
        for i in range(1, b_per):
            pltpu.make_async_copy(
                src, out_hbm.at[b_lo + i, pl.ds(ch_lo, c_per)], sem
            ).start()
        for i in range(1, b_per):
            pltpu.make_async_copy(
                src, out_hbm.at[b_lo + i, pl.ds(ch_lo, c_per)], sem
            ).wait()

    return k


def kernel(token_tensors, row_embed_w, col_embed_w):
    B, _, h, w = token_tensors.shape
    F = row_embed_w.shape[1]
    rowT = row_embed_w.T.reshape(-1)
    colT = col_embed_w.T.reshape(-1)
    return _sc_kernel(B, F, h, w)(rowT, colT)


# speedup vs baseline: 77.8363x; 77.8363x over previous
import functools
import math

import jax
import jax.numpy as jnp
from jax import lax
from jax.experimental import pallas as pl
from jax.experimental.pallas import tpu as pltpu
from jax.experimental.pallas import tpu_sc as plsc


def _sc_kernel(B, F, h, w):
    hw = h * w
    C = 2 * F
    NC, NS, L = 2, 16, 16
    n_cg = 4
    n_bg = NS // (n_cg // NC)
    c_per = C // n_cg
    b_per = B // n_bg
    assert c_per * n_cg == C and b_per * n_bg == B
    assert w % L == 0 and h % L == 0
    tab_stage = c_per * w
    n_x_chunks = w // L
    mesh = plsc.VectorSubcoreMesh(core_axis_name="c", subcore_axis_name="s")

    @functools.partial(
        pl.kernel,
        mesh=mesh,
        out_type=jax.ShapeDtypeStruct((B, C, hw), jnp.float32),
        scratch_types=[
            pltpu.VMEM((tab_stage,), jnp.float32),
            pltpu.VMEM((c_per, hw), jnp.float32),
            pltpu.SemaphoreType.DMA,
        ],
    )
    def k(rowT_hbm, colT_hbm, out_hbm, tab_v, buf_v, sem):
        core = lax.axis_index("c")
        sub = lax.axis_index("s")
        half = sub // n_bg
        bgroup = sub % n_bg
        cgroup = core * NC + half

        t_lo = half * tab_stage

        @pl.when(core == 0)
        def _():
            pltpu.sync_copy(colT_hbm.at[pl.ds(t_lo, tab_stage)],
                            tab_v.at[pl.ds(0, tab_stage)])

        @pl.when(core == 1)
        def _():
            pltpu.sync_copy(rowT_hbm.at[pl.ds(t_lo, tab_stage)],
                            tab_v.at[pl.ds(0, tab_stage)])

        @pl.when(core == 0)
        def _():
            def jloop(j, _):
                chunks = [tab_v[pl.ds(j * w + q * L, L)] for q in range(n_x_chunks)]
                for y in range(h):
                    for q in range(n_x_chunks):
                        buf_v[j, pl.ds(y * w + q * L, L)] = chunks[q]
                return 0

            lax.fori_loop(0, c_per, jloop, 0)

        @pl.when(core == 1)
        def _():
            def jloop(j, _):
                yvecs = [tab_v[pl.ds(j * h + q * L, L)] for q in range(h // L)]
                for y in range(h):
                    val = jnp.full((L,), yvecs[y // L][y % L], jnp.float32)
                    for q in range(n_x_chunks):
                        buf_v[j, pl.ds(y * w + q * L, L)] = val
                return 0

            lax.fori_loop(0, c_per, jloop, 0)

        ch_lo = cgroup * c_per
        b_lo = bgroup * b_per
        for i in range(b_per):
            pltpu.make_async_copy(
                buf_v, out_hbm.at[b_lo + i, pl.ds(ch_lo, c_per), :], sem
            ).start()
        for i in range(b_per):
            pltpu.make_async_copy(
                buf_v, out_hbm.at[b_lo + i, pl.ds(ch_lo, c_per), :], sem
            ).wait()

    return k


def kernel(token_tensors, row_embed_w, col_embed_w):
    B, _, h, w = token_tensors.shape
    F = row_embed_w.shape[1]
    rowT = row_embed_w.T.reshape(-1)
    colT = col_embed_w.T.reshape(-1)
    out = _sc_kernel(B, F, h, w)(rowT, colT)
    return out.reshape(B, 2 * F, h, w)


# speedup vs baseline: 78.0877x; 1.0032x over previous
import functools
import math

import jax
import jax.numpy as jnp
from jax import lax
from jax.experimental import pallas as pl
from jax.experimental.pallas import tpu as pltpu
from jax.experimental.pallas import tpu_sc as plsc


def _sc_kernel(B, F, h, w):
    hw = h * w
    C = 2 * F
    NC, NS, L = 2, 16, 16
    n_cg = 4
    n_bg = NS // (n_cg // NC)
    c_per = C // n_cg
    b_per = B // n_bg
    assert c_per * n_cg == C and b_per * n_bg == B
    assert w % L == 0 and h % L == 0
    tab_stage = c_per * w
    n_x_chunks = w // L
    mesh = plsc.VectorSubcoreMesh(core_axis_name="c", subcore_axis_name="s")

    @functools.partial(
        pl.kernel,
        mesh=mesh,
        out_type=jax.ShapeDtypeStruct((B, C, hw), jnp.float32),
        scratch_types=[
            pltpu.VMEM((tab_stage,), jnp.float32),
            pltpu.VMEM((c_per, hw), jnp.float32),
            pltpu.SemaphoreType.DMA,
        ],
        compiler_params=pltpu.CompilerParams(use_tc_tiling_on_sc=True),
    )
    def k(rowT_hbm, colT_hbm, out_hbm, tab_v, buf_v, sem):
        core = lax.axis_index("c")
        sub = lax.axis_index("s")
        half = sub // n_bg
        bgroup = sub % n_bg
        cgroup = core * NC + half

        t_lo = half * tab_stage

        @pl.when(core == 0)
        def _():
            pltpu.sync_copy(colT_hbm.at[pl.ds(t_lo, tab_stage)],
                            tab_v.at[pl.ds(0, tab_stage)])

        @pl.when(core == 1)
        def _():
            pltpu.sync_copy(rowT_hbm.at[pl.ds(t_lo, tab_stage)],
                            tab_v.at[pl.ds(0, tab_stage)])

        @pl.when(core == 0)
        def _():
            def jloop(j, _):
                chunks = [tab_v[pl.ds(j * w + q * L, L)] for q in range(n_x_chunks)]
                for y in range(h):
                    for q in range(n_x_chunks):
                        buf_v[j, pl.ds(y * w + q * L, L)] = chunks[q]
                return 0

            lax.fori_loop(0, c_per, jloop, 0)

        @pl.when(core == 1)
        def _():
            def jloop(j, _):
                yvecs = [tab_v[pl.ds(j * h + q * L, L)] for q in range(h // L)]
                for y in range(h):
                    val = jnp.full((L,), yvecs[y // L][y % L], jnp.float32)
                    for q in range(n_x_chunks):
                        buf_v[j, pl.ds(y * w + q * L, L)] = val
                return 0

            lax.fori_loop(0, c_per, jloop, 0)

        ch_lo = cgroup * c_per
        b_lo = bgroup * b_per
        for i in range(b_per):
            pltpu.make_async_copy(
                buf_v, out_hbm.at[b_lo + i, pl.ds(ch_lo, c_per), :], sem
            ).start()
        for i in range(b_per):
            pltpu.make_async_copy(
                buf_v, out_hbm.at[b_lo + i, pl.ds(ch_lo, c_per), :], sem
            ).wait()

    return k


def kernel(token_tensors, row_embed_w, col_embed_w):
    B, _, h, w = token_tensors.shape
    F = row_embed_w.shape[1]
    rowT = row_embed_w.T.reshape(-1)
    colT = col_embed_w.T.reshape(-1)
    out = _sc_kernel(B, F, h, w)(rowT, colT)
    return out.reshape(B, 2 * F, h, w)
